# Initial kernel scaffold; baseline (speedup 1.0000x reference)
import functools

import jax
import jax.numpy as jnp
from jax import lax
from jax.experimental import pallas as pl
from jax.experimental.pallas import tpu as pltpu
from jax.experimental.pallas import tpu_sc as plsc

N_NODES = 10000
N_EDGES = 320000
D = 128

NC = 2
NS = 16
NW = NC * NS
EPW = N_EDGES // NW
K = 128
FULL = EPW // K
TAIL = EPW - FULL * K
RPS = N_NODES // NS
DW = 16

_mesh = plsc.VectorSubcoreMesh(core_axis_name="c", subcore_axis_name="s")


@functools.partial(
    pl.kernel,
    out_type=(
        jax.ShapeDtypeStruct((NC, N_NODES, D), jnp.float32),
        jax.ShapeDtypeStruct((NC, N_NODES, DW), jnp.float32),
    ),
    mesh=_mesh,
    scratch_types=[
        pltpu.VMEM((K,), jnp.int32),
        pltpu.VMEM((K,), jnp.int32),
        pltpu.VMEM((K, D), jnp.float32),
        pltpu.VMEM((K, DW), jnp.float32),
        pltpu.VMEM((TAIL,), jnp.int32),
        pltpu.VMEM((TAIL,), jnp.int32),
        pltpu.VMEM((TAIL, D), jnp.float32),
        pltpu.VMEM((TAIL, DW), jnp.float32),
        pltpu.VMEM_SHARED((N_NODES, D), jnp.float32),
        pltpu.VMEM_SHARED((N_NODES, DW), jnp.float32),
        pltpu.SemaphoreType.DMA,
    ],
)
def _sc_aggregate(x_hbm, src_hbm, dst_hbm, zh_hbm, zd_hbm, ones_hbm,
                  ph_hbm, pd_hbm,
                  si_v, di_v, rows_v, ones_v,
                  st_v, dt_v, rowt_v, onet_v,
                  acc_h, acc_d, sem):
    c = lax.axis_index("c")
    s = lax.axis_index("s")
    wid = c * NS + s

    r0 = s * RPS
    pltpu.sync_copy(zh_hbm, acc_h.at[pl.ds(r0, RPS)])
    pltpu.sync_copy(zd_hbm, acc_d.at[pl.ds(r0, RPS)])
    pltpu.sync_copy(ones_hbm, ones_v)
    pltpu.sync_copy(ones_hbm.at[pl.ds(0, TAIL)], onet_v)
    plsc.subcore_barrier()

    base_w = wid * EPW

    @pl.loop(0, FULL)
    def _(j):
        base = pl.multiple_of(base_w + j * K, 8)
        pltpu.sync_copy(src_hbm.at[pl.ds(base, K)], si_v)
        pltpu.sync_copy(dst_hbm.at[pl.ds(base, K)], di_v)
        pltpu.async_copy(x_hbm.at[si_v], rows_v, sem).wait()
        pltpu.sync_copy(rows_v, acc_h.at[di_v], add=True)
        pltpu.sync_copy(ones_v, acc_d.at[di_v], add=True)

    tbase = pl.multiple_of(base_w + FULL * K, 8)
    pltpu.sync_copy(src_hbm.at[pl.ds(tbase, TAIL)], st_v)
    pltpu.sync_copy(dst_hbm.at[pl.ds(tbase, TAIL)], dt_v)
    pltpu.async_copy(x_hbm.at[st_v], rowt_v, sem).wait()
    pltpu.sync_copy(rowt_v, acc_h.at[dt_v], add=True)
    pltpu.sync_copy(onet_v, acc_d.at[dt_v], add=True)

    plsc.subcore_barrier()

    pltpu.sync_copy(acc_h.at[pl.ds(r0, RPS)], ph_hbm.at[c].at[pl.ds(r0, RPS)])
    pltpu.sync_copy(acc_d.at[pl.ds(r0, RPS)], pd_hbm.at[c].at[pl.ds(r0, RPS)])


_ROWS_BLK = 1000


def _tc_body(ph_ref, pd_ref, wt_ref, b_ref, o_ref):
    h = ph_ref[0] + ph_ref[1]
    deg = pd_ref[0][:, :1] + pd_ref[1][:, :1]
    deg = jnp.maximum(deg, 1.0)
    h = h / deg
    o_ref[...] = jnp.dot(h, wt_ref[...], preferred_element_type=jnp.float32) + b_ref[...]


def _tc_linear(ph, pd, wt, b2):
    return pl.pallas_call(
        _tc_body,
        grid=(N_NODES // _ROWS_BLK,),
        in_specs=[
            pl.BlockSpec((NC, _ROWS_BLK, D), lambda i: (0, i, 0)),
            pl.BlockSpec((NC, _ROWS_BLK, DW), lambda i: (0, i, 0)),
            pl.BlockSpec((D, D), lambda i: (0, 0)),
            pl.BlockSpec((1, D), lambda i: (0, 0)),
        ],
        out_specs=pl.BlockSpec((_ROWS_BLK, D), lambda i: (i, 0)),
        out_shape=jax.ShapeDtypeStruct((N_NODES, D), jnp.float32),
    )(ph, pd, wt, b2)


def kernel(inputs, edge_index, W, b):
    src = edge_index[0].astype(jnp.int32)
    dst = edge_index[1].astype(jnp.int32)
    zh = jnp.zeros((RPS, D), jnp.float32)
    zd = jnp.zeros((RPS, DW), jnp.float32)
    ones = jnp.ones((K, DW), jnp.float32)
    ph, pd = _sc_aggregate(inputs, src, dst, zh, zd, ones)
    return _tc_linear(ph, pd, W.T, b.reshape(1, D))


# speedup vs baseline: 6.8365x; 6.8365x over previous
import dataclasses
import functools

import jax
import jax.numpy as jnp
from jax import lax
from jax.experimental import pallas as pl
from jax.experimental.pallas import tpu as pltpu
from jax.experimental.pallas import tpu_sc as plsc

N_NODES = 10000
N_EDGES = 320000
D = 128
L = 16

NC = 2
NS = 16
NW = NC * NS
EPW = N_EDGES // NW
K = 128
FULL = EPW // K
TAIL = EPW - FULL * K
NP = 10240
RPS = NP // NS
DR = NP // D
DRS = DR // NS
CPS = RPS // L

_mesh = plsc.VectorSubcoreMesh(core_axis_name="c", subcore_axis_name="s")

_cp = pltpu.CompilerParams()
if "needs_layout_passes" in pltpu.CompilerParams.__dataclass_fields__:
    _cp = dataclasses.replace(_cp, needs_layout_passes=False)


@functools.partial(
    pl.kernel,
    out_type=(
        jax.ShapeDtypeStruct((NC, NP, D), jnp.float32),
        jax.ShapeDtypeStruct((NC, DR, D), jnp.float32),
    ),
    mesh=_mesh,
    scratch_types=[
        pltpu.VMEM((K,), jnp.int32),
        pltpu.VMEM((K,), jnp.int32),
        pltpu.VMEM((K, D), jnp.float32),
        pltpu.VMEM((TAIL,), jnp.int32),
        pltpu.VMEM((TAIL,), jnp.int32),
        pltpu.VMEM((TAIL, D), jnp.float32),
        pltpu.VMEM((NP,), jnp.float32),
        pltpu.VMEM((RPS,), jnp.float32),
        pltpu.VMEM((DRS, D), jnp.float32),
        pltpu.VMEM_SHARED((NP, D), jnp.float32),
        pltpu.VMEM_SHARED((NS, NP), jnp.float32),
        pltpu.VMEM_SHARED((DR, D), jnp.float32),
        pltpu.SemaphoreType.DMA,
    ],
    compiler_params=_cp,
)
def _sc_aggregate(x_hbm, src_hbm, dst_hbm, zh_hbm, zdl_hbm,
                  ph_hbm, pd_hbm,
                  si_v, di_v, rows_v, st_v, dt_v, rowt_v,
                  dl_v, crow_v, da_v,
                  acc_h, stage_d, deg_img, sem):
    c = lax.axis_index("c")
    s = lax.axis_index("s")
    wid = c * NS + s
    r0 = s * RPS

    pltpu.sync_copy(zh_hbm, acc_h.at[pl.ds(r0, RPS)])
    pltpu.sync_copy(zdl_hbm, dl_v)
    plsc.subcore_barrier()

    ones16 = jnp.full((L,), 1.0, jnp.float32)
    base_w = wid * EPW

    @pl.loop(0, FULL)
    def _(j):
        base = pl.multiple_of(base_w + j * K, 8)
        pltpu.sync_copy(src_hbm.at[pl.ds(base, K)], si_v)
        pltpu.sync_copy(dst_hbm.at[pl.ds(base, K)], di_v)
        pltpu.async_copy(x_hbm.at[si_v], rows_v, sem).wait()
        pltpu.sync_copy(rows_v, acc_h.at[di_v], add=True)
        for t in range(K // L):
            plsc.addupdate_scatter(dl_v, [di_v[pl.ds(t * L, L)]], ones16)

    tbase = pl.multiple_of(base_w + FULL * K, 8)
    pltpu.sync_copy(src_hbm.at[pl.ds(tbase, TAIL)], st_v)
    pltpu.sync_copy(dst_hbm.at[pl.ds(tbase, TAIL)], dt_v)
    pltpu.async_copy(x_hbm.at[st_v], rowt_v, sem).wait()
    pltpu.sync_copy(rowt_v, acc_h.at[dt_v], add=True)
    plsc.addupdate_scatter(dl_v, [dt_v[pl.ds(0, TAIL)]], ones16)

    pltpu.sync_copy(dl_v, stage_d.at[s])
    plsc.subcore_barrier()

    zeros16 = jnp.zeros((L,), jnp.float32)
    for cc in range(CPS):
        da_v[cc // 8, pl.ds((cc % 8) * L, L)] = zeros16

    @pl.loop(0, NS)
    def _(r):
        pltpu.sync_copy(stage_d.at[r].at[pl.ds(r0, RPS)], crow_v)
        for cc in range(CPS):
            sl = pl.ds((cc % 8) * L, L)
            da_v[cc // 8, sl] += crow_v[pl.ds(cc * L, L)]

    pltpu.sync_copy(da_v, deg_img.at[pl.ds(s * DRS, DRS)])
    plsc.subcore_barrier()

    pltpu.sync_copy(acc_h.at[pl.ds(r0, RPS)], ph_hbm.at[c].at[pl.ds(r0, RPS)])

    @pl.when(s == 0)
    def _():
        pltpu.sync_copy(deg_img, pd_hbm.at[c])


_RB = 128


def _tc_body(ph_ref, pd_ref, wt_ref, b_ref, o_ref):
    h = ph_ref[0] + ph_ref[1]
    dr = jnp.maximum(pd_ref[0, 0] + pd_ref[1, 0], 1.0)
    recip = 1.0 / dr
    ri = lax.broadcasted_iota(jnp.int32, (_RB, _RB), 0)
    ci = lax.broadcasted_iota(jnp.int32, (_RB, _RB), 1)
    diag = jnp.where(ri == ci, recip, 0.0)
    hn = jnp.dot(diag, h, preferred_element_type=jnp.float32)
    o_ref[...] = jnp.dot(hn, wt_ref[...], preferred_element_type=jnp.float32) + b_ref[...]


def _tc_linear(ph, pd4, wt, b2):
    return pl.pallas_call(
        _tc_body,
        grid=(pl.cdiv(N_NODES, _RB),),
        in_specs=[
            pl.BlockSpec((NC, _RB, D), lambda i: (0, i, 0)),
            pl.BlockSpec((NC, 1, 1, D), lambda i: (0, i, 0, 0)),
            pl.BlockSpec((D, D), lambda i: (0, 0)),
            pl.BlockSpec((1, D), lambda i: (0, 0)),
        ],
        out_specs=pl.BlockSpec((_RB, D), lambda i: (i, 0)),
        out_shape=jax.ShapeDtypeStruct((N_NODES, D), jnp.float32),
    )(ph, pd4, wt, b2)


def kernel(inputs, edge_index, W, b):
    src = edge_index[0].astype(jnp.int32)
    dst = edge_index[1].astype(jnp.int32)
    zh = jnp.zeros((RPS, D), jnp.float32)
    zdl = jnp.zeros((NP,), jnp.float32)
    ph, pd = _sc_aggregate(inputs, src, dst, zh, zdl)
    pd4 = pd.reshape(NC, DR, 1, D)
    return _tc_linear(ph, pd4, W.T, b.reshape(1, D))


# speedup vs baseline: 9.6208x; 1.4073x over previous
import dataclasses
import functools

import jax
import jax.numpy as jnp
from jax import lax
from jax.experimental import pallas as pl
from jax.experimental.pallas import tpu as pltpu
from jax.experimental.pallas import tpu_sc as plsc

N_NODES = 10000
N_EDGES = 320000
D = 128
L = 16

NC = 2
NS = 16
NW = NC * NS
K = 128
NCHUNK = N_EDGES // K
CPW = NCHUNK // NW
XTRA = NCHUNK - CPW * NW
NP = 10240
RPS = NP // NS
DR = NP // D

_mesh = plsc.VectorSubcoreMesh(core_axis_name="c", subcore_axis_name="s")

_cp = pltpu.CompilerParams()
if "needs_layout_passes" in pltpu.CompilerParams.__dataclass_fields__:
    _cp = dataclasses.replace(_cp, needs_layout_passes=False)


@functools.partial(
    pl.kernel,
    out_type=(
        jax.ShapeDtypeStruct((NC, NP, D), jnp.float32),
        jax.ShapeDtypeStruct((NC, NS, DR, D), jnp.float32),
    ),
    mesh=_mesh,
    scratch_types=[
        pltpu.VMEM((K,), jnp.int32),
        pltpu.VMEM((K,), jnp.int32),
        pltpu.VMEM((K,), jnp.int32),
        pltpu.VMEM((K,), jnp.int32),
        pltpu.VMEM((K, D), jnp.float32),
        pltpu.VMEM((K, D), jnp.float32),
        pltpu.VMEM((DR, D), jnp.float32),
        pltpu.VMEM_SHARED((NP, D), jnp.float32),
        pltpu.SemaphoreType.DMA,
        pltpu.SemaphoreType.DMA,
    ],
    compiler_params=_cp,
)
def _sc_aggregate(x_hbm, src_hbm, dst_hbm, zh_hbm, zd_hbm,
                  ph_hbm, pd_hbm,
                  si0_v, si1_v, di0_v, di1_v, rows0_v, rows1_v, dl_v,
                  acc_h, sem0, sem1):
    c = lax.axis_index("c")
    s = lax.axis_index("s")
    wid = c * NS + s
    r0 = s * RPS
    ebase = wid * CPW * K

    pltpu.sync_copy(zh_hbm, acc_h.at[pl.ds(r0, RPS)])
    pltpu.sync_copy(zd_hbm, dl_v)
    plsc.subcore_barrier()

    ones16 = jnp.full((L,), 1.0, jnp.float32)
    si_bufs = (si0_v, si1_v)
    di_bufs = (di0_v, di1_v)
    row_bufs = (rows0_v, rows1_v)
    sems = (sem0, sem1)

    def fetch(t, b):
        base = pl.multiple_of(ebase + t * K, 8)
        pltpu.sync_copy(src_hbm.at[pl.ds(base, K)], si_bufs[b])
        pltpu.sync_copy(dst_hbm.at[pl.ds(base, K)], di_bufs[b])
        pltpu.async_copy(x_hbm.at[si_bufs[b]], row_bufs[b], sems[b])

    def consume(b):
        pltpu.make_async_copy(x_hbm.at[si_bufs[b]], row_bufs[b],
                              sems[b]).wait()
        pltpu.sync_copy(row_bufs[b], acc_h.at[di_bufs[b]], add=True)
        for t in range(K // L):
            idx16 = di_bufs[b][pl.ds(t * L, L)]
            plsc.addupdate_scatter(
                dl_v, [lax.shift_right_logical(idx16, 7),
                       lax.bitwise_and(idx16, 127)], ones16)

    fetch(0, 0)
    fetch(1, 1)

    @pl.loop(0, CPW // 2)
    def _(jj):
        t0 = jj * 2
        for b in range(2):
            consume(b)

            @pl.when(t0 + b + 2 < CPW)
            def _():
                fetch(t0 + b + 2, b)

    @pl.when(wid < XTRA)
    def _():
        base = pl.multiple_of((NW * CPW + wid) * K, 8)
        pltpu.sync_copy(src_hbm.at[pl.ds(base, K)], si0_v)
        pltpu.sync_copy(dst_hbm.at[pl.ds(base, K)], di0_v)
        pltpu.async_copy(x_hbm.at[si0_v], rows0_v, sem0)
        consume(0)

    plsc.subcore_barrier()

    pltpu.sync_copy(acc_h.at[pl.ds(r0, RPS)], ph_hbm.at[c].at[pl.ds(r0, RPS)])
    pltpu.sync_copy(dl_v, pd_hbm.at[c].at[s])


_RB = 128


def _tc_body(ph_ref, pd_ref, wt_ref, b_ref, o_ref):
    h = ph_ref[0] + ph_ref[1]
    dsum = jnp.sum(pd_ref[...], axis=(0, 1))
    dr = jnp.maximum(dsum[0], 1.0)
    recip = 1.0 / dr
    ri = lax.broadcasted_iota(jnp.int32, (_RB, _RB), 0)
    ci = lax.broadcasted_iota(jnp.int32, (_RB, _RB), 1)
    diag = jnp.where(ri == ci, recip, 0.0)
    hn = jnp.dot(diag, h, preferred_element_type=jnp.float32)
    o_ref[...] = jnp.dot(hn, wt_ref[...], preferred_element_type=jnp.float32) + b_ref[...]


def _tc_linear(ph, pd5, wt, b2):
    return pl.pallas_call(
        _tc_body,
        grid=(pl.cdiv(N_NODES, _RB),),
        in_specs=[
            pl.BlockSpec((NC, _RB, D), lambda i: (0, i, 0)),
            pl.BlockSpec((NC, NS, 1, 1, D), lambda i: (0, 0, i, 0, 0)),
            pl.BlockSpec((D, D), lambda i: (0, 0)),
            pl.BlockSpec((1, D), lambda i: (0, 0)),
        ],
        out_specs=pl.BlockSpec((_RB, D), lambda i: (i, 0)),
        out_shape=jax.ShapeDtypeStruct((N_NODES, D), jnp.float32),
    )(ph, pd5, wt, b2)


def kernel(inputs, edge_index, W, b):
    src = edge_index[0].astype(jnp.int32)
    dst = edge_index[1].astype(jnp.int32)
    zh = jnp.zeros((RPS, D), jnp.float32)
    zd = jnp.zeros((DR, D), jnp.float32)
    ph, pd = _sc_aggregate(inputs, src, dst, zh, zd)
    pd5 = pd.reshape(NC, NS, DR, 1, D)
    return _tc_linear(ph, pd5, W.T, b.reshape(1, D))


# speedup vs baseline: 13.8326x; 1.4378x over previous
import dataclasses
import functools

import jax
import jax.numpy as jnp
from jax import lax
from jax.experimental import pallas as pl
from jax.experimental.pallas import tpu as pltpu
from jax.experimental.pallas import tpu_sc as plsc

N_NODES = 10000
N_EDGES = 320000
D = 128
L = 16

NC = 2
NS = 16
NW = NC * NS
K = 128
NCHUNK = N_EDGES // K
CPW = NCHUNK // NW
XTRA = NCHUNK - CPW * NW
NP = 10240
RPS = NP // NS
DR = NP // D

_mesh = plsc.VectorSubcoreMesh(core_axis_name="c", subcore_axis_name="s")

_cp = pltpu.CompilerParams()
if "needs_layout_passes" in pltpu.CompilerParams.__dataclass_fields__:
    _cp = dataclasses.replace(_cp, needs_layout_passes=False)


@functools.partial(
    pl.kernel,
    out_type=(
        jax.ShapeDtypeStruct((NC, NP, D), jnp.float32),
        jax.ShapeDtypeStruct((NC, NS, DR, D), jnp.float32),
    ),
    mesh=_mesh,
    scratch_types=[
        pltpu.VMEM((2, K), jnp.int32),
        pltpu.VMEM((2, K), jnp.int32),
        pltpu.VMEM((K, D), jnp.float32),
        pltpu.VMEM((K, D), jnp.float32),
        pltpu.VMEM((DR, D), jnp.float32),
        pltpu.VMEM_SHARED((NP, D), jnp.float32),
        pltpu.SemaphoreType.DMA,
        pltpu.SemaphoreType.DMA,
        pltpu.SemaphoreType.DMA,
        pltpu.SemaphoreType.DMA,
    ],
    compiler_params=_cp,
)
def _sc_aggregate(x_hbm, sidi_hbm, zh_hbm, zd_hbm,
                  ph_hbm, pd_hbm,
                  sidi0_v, sidi1_v, rows0_v, rows1_v, dl_v,
                  acc_h, semg0, semg1, sems0, sems1):
    c = lax.axis_index("c")
    s = lax.axis_index("s")
    wid = c * NS + s
    r0 = s * RPS
    cbase = wid * CPW

    pltpu.sync_copy(zh_hbm, acc_h.at[pl.ds(r0, RPS)])
    pltpu.sync_copy(zd_hbm, dl_v)
    plsc.subcore_barrier()

    ones16 = jnp.full((L,), 1.0, jnp.float32)
    sidi_bufs = (sidi0_v, sidi1_v)
    row_bufs = (rows0_v, rows1_v)
    semgs = (semg0, semg1)
    semss = (sems0, sems1)

    def fetch(g, b):
        pltpu.sync_copy(sidi_hbm.at[g], sidi_bufs[b])
        pltpu.async_copy(x_hbm.at[sidi_bufs[b].at[0]], row_bufs[b], semgs[b])

    def consume(b):
        pltpu.make_async_copy(x_hbm.at[sidi_bufs[b].at[0]], row_bufs[b],
                              semgs[b]).wait()
        pltpu.async_copy(row_bufs[b], acc_h.at[sidi_bufs[b].at[1]],
                         semss[b], add=True)
        for t in range(K // L):
            idx16 = sidi_bufs[b][1, pl.ds(t * L, L)]
            plsc.addupdate_scatter(
                dl_v, [lax.shift_right_logical(idx16, 7),
                       lax.bitwise_and(idx16, 127)], ones16)

    def drain_scatter(b):
        pltpu.make_async_copy(row_bufs[b], acc_h.at[sidi_bufs[b].at[1]],
                              semss[b]).wait()

    fetch(cbase, 0)
    fetch(cbase + 1, 1)

    @pl.loop(0, CPW // 2)
    def _(jj):
        t0 = jj * 2
        for b in range(2):
            consume(b)
        for b in range(2):
            @pl.when(t0 + b + 2 < CPW)
            def _():
                drain_scatter(b)
                fetch(cbase + t0 + b + 2, b)

    drain_scatter(0)
    drain_scatter(1)

    @pl.when(wid < XTRA)
    def _():
        fetch(NW * CPW + wid, 0)
        consume(0)
        drain_scatter(0)

    plsc.subcore_barrier()

    pltpu.sync_copy(acc_h.at[pl.ds(r0, RPS)], ph_hbm.at[c].at[pl.ds(r0, RPS)])
    pltpu.sync_copy(dl_v, pd_hbm.at[c].at[s])


_RB = 1024
_SB = _RB // D


def _tc_body(ph_ref, pd_ref, wt_ref, b_ref, o_ref):
    h = ph_ref[0] + ph_ref[1]
    dall = jnp.sum(pd_ref[...], axis=(0, 1))
    ri = lax.broadcasted_iota(jnp.int32, (D, D), 0)
    ci = lax.broadcasted_iota(jnp.int32, (D, D), 1)
    eye = ri == ci
    wt = wt_ref[...]
    bb = b_ref[...]
    for r in range(_SB):
        recip = 1.0 / jnp.maximum(dall[r], 1.0)
        diag = jnp.where(eye, recip, 0.0)
        hr = h[r * D:(r + 1) * D]
        hn = jnp.dot(diag, hr, preferred_element_type=jnp.float32)
        o_ref[pl.ds(r * D, D), :] = (
            jnp.dot(hn, wt, preferred_element_type=jnp.float32) + bb)


def _tc_linear(ph, pd5, wt, b2):
    return pl.pallas_call(
        _tc_body,
        grid=(pl.cdiv(N_NODES, _RB),),
        in_specs=[
            pl.BlockSpec((NC, _RB, D), lambda i: (0, i, 0)),
            pl.BlockSpec((NC, NS, _SB, 1, D), lambda i: (0, 0, i, 0, 0)),
            pl.BlockSpec((D, D), lambda i: (0, 0)),
            pl.BlockSpec((1, D), lambda i: (0, 0)),
        ],
        out_specs=pl.BlockSpec((_RB, D), lambda i: (i, 0)),
        out_shape=jax.ShapeDtypeStruct((N_NODES, D), jnp.float32),
    )(ph, pd5, wt, b2)


def kernel(inputs, edge_index, W, b):
    ei = edge_index.astype(jnp.int32)
    sidi = ei.reshape(2, NCHUNK, K).transpose(1, 0, 2)
    zh = jnp.zeros((RPS, D), jnp.float32)
    zd = jnp.zeros((DR, D), jnp.float32)
    ph, pd = _sc_aggregate(inputs, sidi, zh, zd)
    pd5 = pd.reshape(NC, NS, DR, 1, D)
    return _tc_linear(ph, pd5, W.T, b.reshape(1, D))


# speedup vs baseline: 14.6809x; 1.0613x over previous
import dataclasses
import functools

import jax
import jax.numpy as jnp
from jax import lax
from jax.experimental import pallas as pl
from jax.experimental.pallas import tpu as pltpu
from jax.experimental.pallas import tpu_sc as plsc

N_NODES = 10000
N_EDGES = 320000
D = 128
L = 16

NC = 2
NS = 16
NW = NC * NS
K = 64
NB = 4
NCHUNK = N_EDGES // K
CPW = NCHUNK // NW
XTRA = NCHUNK - CPW * NW
NP = 10240
RPS = NP // NS
DR = NP // D

_mesh = plsc.VectorSubcoreMesh(core_axis_name="c", subcore_axis_name="s")

_cp = pltpu.CompilerParams()
if "needs_layout_passes" in pltpu.CompilerParams.__dataclass_fields__:
    _cp = dataclasses.replace(_cp, needs_layout_passes=False)


@functools.partial(
    pl.kernel,
    out_type=(
        jax.ShapeDtypeStruct((NC, NP, D), jnp.float32),
        jax.ShapeDtypeStruct((NC, NS, DR, D), jnp.float32),
    ),
    mesh=_mesh,
    scratch_types=[
        [pltpu.VMEM((2, K), jnp.int32)] * NB,
        [pltpu.VMEM((K, D), jnp.float32)] * NB,
        pltpu.VMEM((DR, D), jnp.float32),
        pltpu.VMEM_SHARED((NP, D), jnp.float32),
        [pltpu.SemaphoreType.DMA] * NB,
        [pltpu.SemaphoreType.DMA] * NB,
        [pltpu.SemaphoreType.DMA] * NB,
    ],
    compiler_params=_cp,
)
def _sc_aggregate(x_hbm, sidi_hbm, zh_hbm, zd_hbm,
                  ph_hbm, pd_hbm,
                  sidi, rows, dl_v, acc_h, semi, semg, sems):
    c = lax.axis_index("c")
    s = lax.axis_index("s")
    wid = c * NS + s
    r0 = s * RPS
    cbase = wid * CPW

    pltpu.sync_copy(zh_hbm, acc_h.at[pl.ds(r0, RPS)])
    pltpu.sync_copy(zd_hbm, dl_v)
    plsc.subcore_barrier()

    ones16 = jnp.full((L,), 1.0, jnp.float32)

    def start_idx(g, b):
        pltpu.async_copy(sidi_hbm.at[g], sidi[b], semi[b])

    def wait_idx(b):
        pltpu.make_async_copy(sidi_hbm.at[0], sidi[b], semi[b]).wait()

    def start_gather(b):
        pltpu.async_copy(x_hbm.at[sidi[b].at[0]], rows[b], semg[b])

    def wait_gather(b):
        pltpu.make_async_copy(x_hbm.at[sidi[b].at[0]], rows[b],
                              semg[b]).wait()

    def start_scatter(b):
        pltpu.async_copy(rows[b], acc_h.at[sidi[b].at[1]], sems[b], add=True)

    def drain_scatter(b):
        pltpu.make_async_copy(rows[b], acc_h.at[sidi[b].at[1]],
                              sems[b]).wait()

    def deg(b):
        for t in range(K // L):
            idx16 = sidi[b][1, pl.ds(t * L, L)]
            plsc.addupdate_scatter(
                dl_v, [lax.shift_right_logical(idx16, 7),
                       lax.bitwise_and(idx16, 127)], ones16)

    start_idx(cbase, 0)
    start_idx(cbase + 1, 1)
    start_idx(cbase + 2, 2)
    wait_idx(0)
    start_gather(0)
    wait_idx(1)
    start_gather(1)

    @pl.loop(0, CPW // NB)
    def _(jj):
        t0 = jj * NB
        for b in range(NB):
            t = t0 + b

            @pl.when(t >= 1)
            def _():
                drain_scatter((b + NB - 1) % NB)

            @pl.when(t + 3 < CPW)
            def _():
                start_idx(cbase + t + 3, (b + 3) % NB)

            @pl.when(t + 2 < CPW)
            def _():
                wait_idx((b + 2) % NB)
                start_gather((b + 2) % NB)

            wait_gather(b)
            start_scatter(b)
            deg(b)

    drain_scatter((CPW - 1) % NB)

    @pl.when(wid < XTRA)
    def _():
        g = NW * CPW + wid
        start_idx(g, 0)
        wait_idx(0)
        start_gather(0)
        wait_gather(0)
        pltpu.sync_copy(rows[0], acc_h.at[sidi[0].at[1]], add=True)
        deg(0)

    plsc.subcore_barrier()

    pltpu.sync_copy(acc_h.at[pl.ds(r0, RPS)], ph_hbm.at[c].at[pl.ds(r0, RPS)])
    pltpu.sync_copy(dl_v, pd_hbm.at[c].at[s])


_RB = 1024
_SB = _RB // D


def _tc_body(ph_ref, pd_ref, wt_ref, b_ref, o_ref):
    h = ph_ref[0] + ph_ref[1]
    dall = jnp.sum(pd_ref[...], axis=(0, 1))
    ri = lax.broadcasted_iota(jnp.int32, (D, D), 0)
    ci = lax.broadcasted_iota(jnp.int32, (D, D), 1)
    eye = ri == ci
    wt = wt_ref[...]
    bb = b_ref[...]
    for r in range(_SB):
        recip = 1.0 / jnp.maximum(dall[r], 1.0)
        diag = jnp.where(eye, recip, 0.0)
        hr = h[r * D:(r + 1) * D]
        hn = jnp.dot(diag, hr, preferred_element_type=jnp.float32)
        o_ref[pl.ds(r * D, D), :] = (
            jnp.dot(hn, wt, preferred_element_type=jnp.float32) + bb)


def _tc_linear(ph, pd5, wt, b2):
    return pl.pallas_call(
        _tc_body,
        grid=(pl.cdiv(N_NODES, _RB),),
        in_specs=[
            pl.BlockSpec((NC, _RB, D), lambda i: (0, i, 0)),
            pl.BlockSpec((NC, NS, _SB, 1, D), lambda i: (0, 0, i, 0, 0)),
            pl.BlockSpec((D, D), lambda i: (0, 0)),
            pl.BlockSpec((1, D), lambda i: (0, 0)),
        ],
        out_specs=pl.BlockSpec((_RB, D), lambda i: (i, 0)),
        out_shape=jax.ShapeDtypeStruct((N_NODES, D), jnp.float32),
    )(ph, pd5, wt, b2)


def kernel(inputs, edge_index, W, b):
    ei = edge_index.astype(jnp.int32)
    sidi = ei.reshape(2, NCHUNK, K).transpose(1, 0, 2)
    zh = jnp.zeros((RPS, D), jnp.float32)
    zd = jnp.zeros((DR, D), jnp.float32)
    ph, pd = _sc_aggregate(inputs, sidi, zh, zd)
    pd5 = pd.reshape(NC, NS, DR, 1, D)
    return _tc_linear(ph, pd5, W.T, b.reshape(1, D))
